# Initial kernel scaffold; baseline (speedup 1.0000x reference)
import jax
import jax.numpy as jnp
from jax.experimental import pallas as pl


def kernel(x, weight):
    raise NotImplementedError("write your pallas kernel here")


# speedup vs baseline: 1.5481x; 1.5481x over previous
import functools

import jax
import jax.numpy as jnp
from jax import lax
from jax.experimental import pallas as pl
from jax.experimental.pallas import tpu as pltpu
from jax.experimental.pallas import tpu_sc as plsc

_CHUNK = 1024


@functools.partial(jax.jit, static_argnames=())
def _embedding_lookup(idx_flat, weight):
    n = idx_flat.shape[0]
    V, D = weight.shape
    info = plsc.get_sparse_core_info()
    NC, NS = info.num_cores, info.num_subcores
    NW = NC * NS
    assert n % NW == 0
    b_per_w = n // NW
    C = _CHUNK
    assert b_per_w % C == 0
    n_chunks = b_per_w // C

    mesh = plsc.VectorSubcoreMesh(core_axis_name="c", subcore_axis_name="s")

    @functools.partial(
        pl.kernel,
        mesh=mesh,
        out_type=jax.ShapeDtypeStruct((n, D), jnp.float32),
        scratch_types=[
            pltpu.VMEM((C,), jnp.int32),
            pltpu.VMEM((C, D), jnp.float32),
            pltpu.SemaphoreType.DMA,
        ],
        compiler_params=pltpu.CompilerParams(use_tc_tiling_on_sc=False),
    )
    def emb(table_hbm, idx_hbm, out_hbm, idx_v, rows_v, sem):
        wid = lax.axis_index("s") * NC + lax.axis_index("c")
        base = wid * b_per_w

        def body(j, carry):
            off = base + j * C
            pltpu.sync_copy(idx_hbm.at[pl.ds(off, C)], idx_v)
            pltpu.async_copy(table_hbm.at[idx_v], rows_v, sem).wait()
            pltpu.sync_copy(rows_v, out_hbm.at[pl.ds(off, C)])
            return carry

        lax.fori_loop(0, n_chunks, body, 0)

    return emb(weight, idx_flat)


def kernel(x, weight):
    B, F = x.shape
    out = _embedding_lookup(x.reshape(B * F).astype(jnp.int32), weight)
    return out.reshape(B, F, weight.shape[1])


# speedup vs baseline: 1.5778x; 1.0191x over previous
import functools

import jax
import jax.numpy as jnp
from jax import lax
from jax.experimental import pallas as pl
from jax.experimental.pallas import tpu as pltpu
from jax.experimental.pallas import tpu_sc as plsc

_NBUF = 4
_CHUNK = 832


@jax.jit
def _embedding_lookup(idx_flat, weight):
    n = idx_flat.shape[0]
    V, D = weight.shape
    info = plsc.get_sparse_core_info()
    NC, NS = info.num_cores, info.num_subcores
    NW = NC * NS
    assert n % NW == 0
    b_per_w = n // NW
    C = _CHUNK
    NBUF = _NBUF
    assert b_per_w % (C * NBUF) == 0
    n_chunks = b_per_w // C

    mesh = plsc.VectorSubcoreMesh(core_axis_name="c", subcore_axis_name="s")

    @functools.partial(
        pl.kernel,
        mesh=mesh,
        out_type=jax.ShapeDtypeStruct((n, D), jnp.float32),
        scratch_types=[
            pltpu.VMEM((b_per_w,), jnp.int32),
            *[pltpu.VMEM((C, D), jnp.float32) for _ in range(NBUF)],
            *[pltpu.SemaphoreType.DMA for _ in range(2 * NBUF)],
        ],
        compiler_params=pltpu.CompilerParams(use_tc_tiling_on_sc=False),
    )
    def emb(table_hbm, idx_hbm, out_hbm, idx_v, *bufs_and_sems):
        rows = bufs_and_sems[:NBUF]
        gsem = bufs_and_sems[NBUF : 2 * NBUF]
        ssem = bufs_and_sems[2 * NBUF :]
        wid = lax.axis_index("s") * NC + lax.axis_index("c")
        base = wid * b_per_w

        def gather(j, b):
            return pltpu.make_async_copy(
                table_hbm.at[idx_v.at[pl.ds(j * C, C)]], rows[b], gsem[b]
            )

        def store(j, b):
            return pltpu.make_async_copy(
                rows[b], out_hbm.at[pl.ds(base + j * C, C)], ssem[b]
            )

        pltpu.sync_copy(idx_hbm.at[pl.ds(base, b_per_w)], idx_v)

        for b in range(NBUF):
            gather(b, b).start()

        def step(g, carry):
            for b in range(NBUF):
                j = g * NBUF + b
                gather(j, b).wait()
                store(j, b).start()
                jn = j + NBUF

                @pl.when(jn < n_chunks)
                def _():
                    store(j, b).wait()
                    gather(jn, b).start()

            return carry

        lax.fori_loop(0, n_chunks // NBUF, step, 0)

        for b in range(NBUF):
            store(n_chunks - NBUF + b, b).wait()

    return emb(weight, idx_flat)


def kernel(x, weight):
    B, F = x.shape
    out = _embedding_lookup(x.reshape(B * F).astype(jnp.int32), weight)
    return out.reshape(B, F, weight.shape[1])
